# Initial kernel scaffold; baseline (speedup 1.0000x reference)
import jax
import jax.numpy as jnp
from jax.experimental import pallas as pl


def kernel(x, edge_index, edge_attr, W, b_lin, We, be, bias):
    raise NotImplementedError("write your pallas kernel here")


# speedup vs baseline: 8.4428x; 8.4428x over previous
import functools

import jax
import jax.numpy as jnp
from jax import lax
from jax.experimental import pallas as pl
from jax.experimental.pallas import tpu as pltpu
from jax.experimental.pallas import tpu_sc as plsc

N = 10000
E = 160000
D = 256
DH = 128
NS = 16
N_PAD = 10240
EPT = E // NS
CH = 80
NCHUNK = EPT // CH
B_CH = 25
NBLK = NCHUNK // B_CH


def _h_body(x_ref, w_ref, b_ref, o_ref):
    o_ref[0] = lax.dot_general(
        x_ref[...], w_ref[0], (((1,), (1,)), ((), ())),
        preferred_element_type=jnp.float32) + b_ref[0]


def _h_call(x, w3, b3):
    bn = 2000
    return pl.pallas_call(
        _h_body,
        grid=(2, N // bn),
        in_specs=[
            pl.BlockSpec((bn, D), lambda s, i: (i, 0)),
            pl.BlockSpec((1, DH, D), lambda s, i: (s, 0, 0)),
            pl.BlockSpec((1, 1, DH), lambda s, i: (s, 0, 0)),
        ],
        out_specs=pl.BlockSpec((1, bn, DH), lambda s, i: (s, i, 0)),
        out_shape=jax.ShapeDtypeStruct((2, N, DH), jnp.float32),
    )(x, w3, b3)


def _ew_body(ea_ref, we_ref, be_ref, o_ref):
    wbar = jnp.mean(we_ref[...], axis=0)
    bebar = jnp.mean(be_ref[...])
    logits = jnp.sum(ea_ref[...] * wbar, axis=-1) + bebar
    o_ref[...] = jnp.clip(jax.nn.sigmoid(logits), 0.0001, 1.0)


def _ew_call(ea3, we, be2):
    br, bc = 8, 800
    rows = E // bc
    return pl.pallas_call(
        _ew_body,
        grid=(rows // br,),
        in_specs=[
            pl.BlockSpec((br, bc, D), lambda i: (i, 0, 0)),
            pl.BlockSpec((D, D), lambda i: (0, 0)),
            pl.BlockSpec((1, D), lambda i: (0, 0)),
        ],
        out_specs=pl.BlockSpec((br, bc), lambda i: (i, 0)),
        out_shape=jax.ShapeDtypeStruct((rows, bc), jnp.float32),
    )(ea3, we, be2)


def _rsqrt16(d):
    xi = plsc.bitcast(d, jnp.int32)
    yi = jnp.int32(0x5F3759DF) - lax.shift_right_logical(xi, 1)
    y = plsc.bitcast(yi, jnp.float32)
    for _ in range(3):
        y = y * (1.5 - 0.5 * d * y * y)
    return y


_SC_MESH = plsc.VectorSubcoreMesh(
    core_axis_name="c", subcore_axis_name="s", num_cores=2, num_subcores=NS)


@functools.partial(
    pl.kernel,
    out_type=jax.ShapeDtypeStruct((2 * N_PAD, DH), jnp.float32),
    mesh=_SC_MESH,
    scratch_types=[
        pltpu.VMEM((80, 128), jnp.float32),
        pltpu.VMEM((CH, DH), jnp.float32),
        pltpu.VMEM((B_CH, CH), jnp.int32),
        pltpu.VMEM((B_CH, CH), jnp.int32),
        pltpu.VMEM((B_CH, CH), jnp.float32),
        pltpu.VMEM((8, 128), jnp.float32),
        pltpu.VMEM((80,), jnp.int32),
        pltpu.VMEM((1, DH), jnp.float32),
        pltpu.VMEM_SHARED((80, 128), jnp.float32),
        pltpu.VMEM_SHARED((80, 128), jnp.float32),
        pltpu.VMEM_SHARED((N_PAD, DH), jnp.float32),
        pltpu.SemaphoreType.DMA,
    ],
    compiler_params=pltpu.CompilerParams(needs_layout_passes=False),
)
def _sc_kernel(ha_hbm, hb_hbm, row_hbm, col_hbm, ew_hbm, bias_hbm, out_hbm,
               dd_v, buf, rowb, colb, ewb, dtmp, iota80, bias_v,
               deg_sh, dis_sh, acc_sh, sem):
    cid = lax.axis_index("c")
    sid = lax.axis_index("s")
    zero16 = jnp.zeros((16,), jnp.float32)

    pltpu.sync_copy(bias_hbm.at[cid], bias_v)

    def _zrow(r, _):
        for j in range(8):
            dd_v[r, pl.ds(j * 16, 16)] = zero16
        return 0
    lax.fori_loop(0, 80, _zrow, 0)

    def _blk_a(b, _):
        blk = sid * NBLK + b
        pltpu.sync_copy(col_hbm.at[blk], colb)
        pltpu.sync_copy(ew_hbm.at[blk], ewb)

        def _ch(kk, _):
            for p in range(5):
                c16 = colb[kk, pl.ds(p * 16, 16)]
                w16 = ewb[kk, pl.ds(p * 16, 16)]
                plsc.addupdate_scatter(
                    dd_v, [lax.shift_right_logical(c16, 7),
                           lax.bitwise_and(c16, 127)], w16)
            return 0
        lax.fori_loop(0, B_CH, _ch, 0)
        return 0
    lax.fori_loop(0, NBLK, _blk_a, 0)

    def _iota(i, _):
        iota80[pl.ds(i * 16, 16)] = lax.iota(jnp.int32, 16) + i * 16
        return 0
    lax.fori_loop(0, 5, _iota, 0)

    @pl.when(sid == 0)
    def _():
        pltpu.sync_copy(dd_v, deg_sh)
    plsc.subcore_barrier()

    @pl.when(sid != 0)
    def _():
        pltpu.sync_copy(dd_v, deg_sh.at[iota80], add=True)
    plsc.subcore_barrier()

    @pl.when(sid < 10)
    def _():
        r0 = sid * 8
        pltpu.sync_copy(deg_sh.at[pl.ds(r0, 8)], dtmp)
        for i in range(8):
            for j in range(8):
                d = dtmp[i, pl.ds(j * 16, 16)]
                y = _rsqrt16(d)
                dtmp[i, pl.ds(j * 16, 16)] = jnp.where(d > 0.0, y, 0.0)
        pltpu.sync_copy(dtmp, dis_sh.at[pl.ds(r0, 8)])
    plsc.subcore_barrier()
    pltpu.sync_copy(dis_sh, dd_v)

    def _zbuf(r, _):
        for j in range(8):
            buf[r, pl.ds(j * 16, 16)] = zero16
        return 0
    lax.fori_loop(0, CH, _zbuf, 0)
    for k in range(8):
        pltpu.sync_copy(buf, acc_sh.at[pl.ds(sid * 640 + k * 80, 80)])
    plsc.subcore_barrier()

    def _blk_b(b, _):
        blk = sid * NBLK + b
        pltpu.sync_copy(row_hbm.at[blk], rowb)
        pltpu.sync_copy(col_hbm.at[blk], colb)
        pltpu.sync_copy(ew_hbm.at[blk], ewb)

        def _ch(kk, _):
            idx = rowb.at[kk]

            @pl.when(cid == 0)
            def _():
                pltpu.async_copy(ha_hbm.at[idx], buf, sem)

            @pl.when(cid != 0)
            def _():
                pltpu.async_copy(hb_hbm.at[idx], buf, sem)

            scales = []
            for p in range(5):
                r16 = rowb[kk, pl.ds(p * 16, 16)]
                c16 = colb[kk, pl.ds(p * 16, 16)]
                dr = plsc.load_gather(
                    dd_v, [lax.shift_right_logical(r16, 7),
                           lax.bitwise_and(r16, 127)])
                dc = plsc.load_gather(
                    dd_v, [lax.shift_right_logical(c16, 7),
                           lax.bitwise_and(c16, 127)])
                scales.append(ewb[kk, pl.ds(p * 16, 16)] * dr * dc)

            pltpu.make_async_copy(ha_hbm.at[idx], buf, sem).wait()

            for p in range(5):
                s16 = scales[p]
                for i in range(16):
                    s = s16[i]
                    for j in range(8):
                        buf[p * 16 + i, pl.ds(j * 16, 16)] = (
                            buf[p * 16 + i, pl.ds(j * 16, 16)] * s)
            pltpu.sync_copy(buf, acc_sh.at[colb.at[kk]], add=True)
            return 0
        lax.fori_loop(0, B_CH, _ch, 0)
        return 0
    lax.fori_loop(0, NBLK, _blk_b, 0)
    plsc.subcore_barrier()

    nb0 = sid * 640
    offo = cid * N_PAD

    def _wb(r, _):
        pltpu.sync_copy(acc_sh.at[pl.ds(nb0 + r * CH, CH)], buf)

        def _wrow(i, _):
            for j in range(8):
                v = buf[i, pl.ds(j * 16, 16)] + bias_v[0, pl.ds(j * 16, 16)]
                buf[i, pl.ds(j * 16, 16)] = jnp.maximum(v, 0.0)
            return 0
        lax.fori_loop(0, CH, _wrow, 0)
        pltpu.sync_copy(buf, out_hbm.at[pl.ds(offo + nb0 + r * CH, CH)])
        return 0
    lax.fori_loop(0, 8, _wb, 0)


def kernel(x, edge_index, edge_attr, W, b_lin, We, be, bias):
    row = edge_index[0]
    col = edge_index[1]
    h2 = _h_call(x, W.reshape(2, DH, D), b_lin.reshape(2, 1, DH))
    ew = _ew_call(edge_attr.reshape(E // 800, 800, D), We,
                  be.reshape(1, D)).reshape(E)
    outf = _sc_kernel(h2[0], h2[1], row.reshape(NS * NBLK, B_CH, CH),
                      col.reshape(NS * NBLK, B_CH, CH),
                      ew.reshape(NS * NBLK, B_CH, CH),
                      bias.reshape(2, 1, DH))
    return jnp.concatenate([outf[:N], outf[N_PAD:N_PAD + N]], axis=1)


# speedup vs baseline: 9.5285x; 1.1286x over previous
import functools

import jax
import jax.numpy as jnp
from jax import lax
from jax.experimental import pallas as pl
from jax.experimental.pallas import tpu as pltpu
from jax.experimental.pallas import tpu_sc as plsc

N = 10000
E = 160000
D = 256
DH = 128
NS = 16
N_PAD = 10240
EPT = E // NS
CH = 80
NCHUNK = EPT // CH
B_CH = 25
NBLK = NCHUNK // B_CH


def _h_body(x_ref, w_ref, b_ref, o_ref):
    o_ref[0] = lax.dot_general(
        x_ref[...], w_ref[0], (((1,), (1,)), ((), ())),
        preferred_element_type=jnp.float32) + b_ref[0]


def _h_call(x, w3, b3):
    bn = 2000
    return pl.pallas_call(
        _h_body,
        grid=(2, N // bn),
        in_specs=[
            pl.BlockSpec((bn, D), lambda s, i: (i, 0)),
            pl.BlockSpec((1, DH, D), lambda s, i: (s, 0, 0)),
            pl.BlockSpec((1, 1, DH), lambda s, i: (s, 0, 0)),
        ],
        out_specs=pl.BlockSpec((1, bn, DH), lambda s, i: (s, i, 0)),
        out_shape=jax.ShapeDtypeStruct((2, N, DH), jnp.float32),
    )(x, w3, b3)


def _ew_body(ea_ref, we_ref, be_ref, o_ref):
    wbar = jnp.mean(we_ref[...], axis=0)
    bebar = jnp.mean(be_ref[...])
    logits = jnp.sum(ea_ref[...] * wbar, axis=-1) + bebar
    o_ref[...] = jnp.clip(jax.nn.sigmoid(logits), 0.0001, 1.0)


def _ew_call(ea3, we, be2):
    br, bc = 8, 800
    rows = E // bc
    return pl.pallas_call(
        _ew_body,
        grid=(rows // br,),
        in_specs=[
            pl.BlockSpec((br, bc, D), lambda i: (i, 0, 0)),
            pl.BlockSpec((D, D), lambda i: (0, 0)),
            pl.BlockSpec((1, D), lambda i: (0, 0)),
        ],
        out_specs=pl.BlockSpec((br, bc), lambda i: (i, 0)),
        out_shape=jax.ShapeDtypeStruct((rows, bc), jnp.float32),
    )(ea3, we, be2)


def _rsqrt16(d):
    xi = plsc.bitcast(d, jnp.int32)
    yi = jnp.int32(0x5F3759DF) - lax.shift_right_logical(xi, 1)
    y = plsc.bitcast(yi, jnp.float32)
    for _ in range(3):
        y = y * (1.5 - 0.5 * d * y * y)
    return y


_SC_MESH = plsc.VectorSubcoreMesh(
    core_axis_name="c", subcore_axis_name="s", num_cores=2, num_subcores=NS)


@functools.partial(
    pl.kernel,
    out_type=jax.ShapeDtypeStruct((2 * N_PAD, DH), jnp.float32),
    mesh=_SC_MESH,
    scratch_types=[
        pltpu.VMEM((80, 128), jnp.float32),
        pltpu.VMEM((CH, DH), jnp.float32),
        pltpu.VMEM((CH, DH), jnp.float32),
        pltpu.VMEM((B_CH, CH), jnp.int32),
        pltpu.VMEM((B_CH, CH), jnp.int32),
        pltpu.VMEM((B_CH, CH), jnp.float32),
        pltpu.VMEM((80,), jnp.int32),
        pltpu.VMEM((1, DH), jnp.float32),
        pltpu.VMEM_SHARED((80, 128), jnp.float32),
        pltpu.VMEM_SHARED((80, 128), jnp.float32),
        pltpu.VMEM_SHARED((N_PAD, DH), jnp.float32),
        pltpu.SemaphoreType.DMA,
        pltpu.SemaphoreType.DMA,
        pltpu.SemaphoreType.DMA,
        pltpu.SemaphoreType.DMA,
    ],
    compiler_params=pltpu.CompilerParams(needs_layout_passes=False),
)
def _sc_kernel(ha_hbm, hb_hbm, row_hbm, col_hbm, ew_hbm, bias_hbm, out_hbm,
               dd_v, buf0, buf1, rowb, colb, ewb, iota80, bias_v,
               deg_sh, dis_sh, acc_sh, sem_g0, sem_g1, sem_s0, sem_s1):
    cid = lax.axis_index("c")
    sid = lax.axis_index("s")
    zero16 = jnp.zeros((16,), jnp.float32)

    pltpu.sync_copy(bias_hbm.at[cid], bias_v)

    def _zrow(r, _):
        for j in range(8):
            dd_v[r, pl.ds(j * 16, 16)] = zero16
        return 0
    lax.fori_loop(0, 80, _zrow, 0)

    def _blk_a(b, _):
        blk = sid * NBLK + b
        pltpu.sync_copy(col_hbm.at[blk], colb)
        pltpu.sync_copy(ew_hbm.at[blk], ewb)

        def _ch(kk, _):
            for p in range(5):
                c16 = colb[kk, pl.ds(p * 16, 16)]
                w16 = ewb[kk, pl.ds(p * 16, 16)]
                plsc.addupdate_scatter(
                    dd_v, [lax.shift_right_logical(c16, 7),
                           lax.bitwise_and(c16, 127)], w16)
            return 0
        lax.fori_loop(0, B_CH, _ch, 0)
        return 0
    lax.fori_loop(0, NBLK, _blk_a, 0)

    def _iota(i, _):
        iota80[pl.ds(i * 16, 16)] = lax.iota(jnp.int32, 16) + i * 16
        return 0
    lax.fori_loop(0, 5, _iota, 0)

    @pl.when(sid == 0)
    def _():
        pltpu.sync_copy(dd_v, deg_sh)
    plsc.subcore_barrier()

    @pl.when(sid != 0)
    def _():
        pltpu.sync_copy(dd_v, deg_sh.at[iota80], add=True)
    plsc.subcore_barrier()

    @pl.when(sid < 10)
    def _():
        r0 = sid * 8
        pltpu.sync_copy(deg_sh.at[pl.ds(r0, 8)], buf1.at[pl.ds(0, 8)])
        for i in range(8):
            for j in range(8):
                d = buf1[i, pl.ds(j * 16, 16)]
                y = _rsqrt16(d)
                buf1[i, pl.ds(j * 16, 16)] = jnp.where(d > 0.0, y, 0.0)
        pltpu.sync_copy(buf1.at[pl.ds(0, 8)], dis_sh.at[pl.ds(r0, 8)])
    plsc.subcore_barrier()
    pltpu.sync_copy(dis_sh, dd_v)

    def _zbuf(r, _):
        for j in range(8):
            buf0[r, pl.ds(j * 16, 16)] = zero16
        return 0
    lax.fori_loop(0, CH, _zbuf, 0)
    for k in range(8):
        pltpu.sync_copy(buf0, acc_sh.at[pl.ds(sid * 640 + k * 80, 80)])
    plsc.subcore_barrier()

    def _issue(idx_row, dstbuf, sem):
        @pl.when(cid == 0)
        def _():
            pltpu.async_copy(ha_hbm.at[idx_row], dstbuf, sem)

        @pl.when(cid != 0)
        def _():
            pltpu.async_copy(hb_hbm.at[idx_row], dstbuf, sem)

    def _scales(kk):
        out = []
        for p in range(5):
            r16 = rowb[kk, pl.ds(p * 16, 16)]
            c16 = colb[kk, pl.ds(p * 16, 16)]
            dr = plsc.load_gather(
                dd_v, [lax.shift_right_logical(r16, 7),
                       lax.bitwise_and(r16, 127)])
            dc = plsc.load_gather(
                dd_v, [lax.shift_right_logical(c16, 7),
                       lax.bitwise_and(c16, 127)])
            out.append(ewb[kk, pl.ds(p * 16, 16)] * dr * dc)
        return out

    def _scale_rows(bufx, scales):
        for p in range(5):
            s16 = scales[p]
            for i in range(16):
                s = s16[i]
                for j in range(8):
                    bufx[p * 16 + i, pl.ds(j * 16, 16)] = (
                        bufx[p * 16 + i, pl.ds(j * 16, 16)] * s)

    def _blk_b(b, _):
        @pl.when(b > 0)
        def _():
            pltpu.make_async_copy(buf0, acc_sh.at[colb.at[0]], sem_s0).wait()
            pltpu.make_async_copy(buf1, acc_sh.at[colb.at[0]], sem_s1).wait()
        blk = sid * NBLK + b
        pltpu.sync_copy(row_hbm.at[blk], rowb)
        pltpu.sync_copy(col_hbm.at[blk], colb)
        pltpu.sync_copy(ew_hbm.at[blk], ewb)
        _issue(rowb.at[0], buf0, sem_g0)

        def _pair(p, _):
            k0 = 2 * p
            k1 = 2 * p + 1

            @pl.when(p > 0)
            def _():
                pltpu.make_async_copy(
                    buf1, acc_sh.at[colb.at[k1]], sem_s1).wait()
            _issue(rowb.at[k1], buf1, sem_g1)
            sc0 = _scales(k0)
            pltpu.make_async_copy(ha_hbm.at[rowb.at[k0]], buf0, sem_g0).wait()
            _scale_rows(buf0, sc0)
            pltpu.async_copy(buf0, acc_sh.at[colb.at[k0]], sem_s0, add=True)
            sc1 = _scales(k1)
            pltpu.make_async_copy(ha_hbm.at[rowb.at[k1]], buf1, sem_g1).wait()
            _scale_rows(buf1, sc1)
            pltpu.async_copy(buf1, acc_sh.at[colb.at[k1]], sem_s1, add=True)
            pltpu.make_async_copy(buf0, acc_sh.at[colb.at[k0]], sem_s0).wait()
            _issue(rowb.at[k0 + 2], buf0, sem_g0)
            return 0
        lax.fori_loop(0, (B_CH - 1) // 2, _pair, 0)

        sc = _scales(B_CH - 1)
        pltpu.make_async_copy(
            ha_hbm.at[rowb.at[B_CH - 1]], buf0, sem_g0).wait()
        _scale_rows(buf0, sc)
        pltpu.async_copy(
            buf0, acc_sh.at[colb.at[B_CH - 1]], sem_s0, add=True)
        return 0
    lax.fori_loop(0, NBLK, _blk_b, 0)
    pltpu.make_async_copy(buf0, acc_sh.at[colb.at[0]], sem_s0).wait()
    pltpu.make_async_copy(buf1, acc_sh.at[colb.at[0]], sem_s1).wait()
    plsc.subcore_barrier()

    nb0 = sid * 640
    offo = cid * N_PAD

    def _wb(r, _):
        pltpu.sync_copy(acc_sh.at[pl.ds(nb0 + r * CH, CH)], buf0)

        def _wrow(i, _):
            for j in range(8):
                v = buf0[i, pl.ds(j * 16, 16)] + bias_v[0, pl.ds(j * 16, 16)]
                buf0[i, pl.ds(j * 16, 16)] = jnp.maximum(v, 0.0)
            return 0
        lax.fori_loop(0, CH, _wrow, 0)
        pltpu.sync_copy(buf0, out_hbm.at[pl.ds(offo + nb0 + r * CH, CH)])
        return 0
    lax.fori_loop(0, 8, _wb, 0)


def kernel(x, edge_index, edge_attr, W, b_lin, We, be, bias):
    row = edge_index[0]
    col = edge_index[1]
    h2 = _h_call(x, W.reshape(2, DH, D), b_lin.reshape(2, 1, DH))
    ew = _ew_call(edge_attr.reshape(E // 800, 800, D), We,
                  be.reshape(1, D)).reshape(E)
    outf = _sc_kernel(h2[0], h2[1], row.reshape(NS * NBLK, B_CH, CH),
                      col.reshape(NS * NBLK, B_CH, CH),
                      ew.reshape(NS * NBLK, B_CH, CH),
                      bias.reshape(2, 1, DH))
    return jnp.concatenate([outf[:N], outf[N_PAD:N_PAD + N]], axis=1)


# speedup vs baseline: 10.2192x; 1.0725x over previous
import functools

import jax
import jax.numpy as jnp
from jax import lax
from jax.experimental import pallas as pl
from jax.experimental.pallas import tpu as pltpu
from jax.experimental.pallas import tpu_sc as plsc

N = 10000
E = 160000
D = 256
DH = 128
NS = 16
N_PAD = 10240
EPT = E // NS
CH = 80
NCHUNK = EPT // CH
B_CH = 25
NBLK = NCHUNK // B_CH
NWCH = N // CH


def _tc_body(x_ref, ea_ref, w_ref, b_ref, we_ref, be_ref,
             ha_ref, hb_ref, ew_ref):
    h = lax.dot_general(
        x_ref[...], w_ref[...], (((1,), (1,)), ((), ())),
        preferred_element_type=jnp.float32) + b_ref[0]
    ha_ref[...] = h[:, :DH]
    hb_ref[...] = h[:, DH:]
    wbar = jnp.mean(we_ref[...], axis=0)
    bebar = jnp.mean(be_ref[...])
    logits = jnp.sum(ea_ref[...] * wbar, axis=-1) + bebar
    ew_ref[...] = jnp.clip(jax.nn.sigmoid(logits), 0.0001, 1.0)


def _tc_call(x, ea3, w, b2, we, be2):
    g = 25
    bn = N // g
    br = (E // 800) // g
    ha, hb, ew = pl.pallas_call(
        _tc_body,
        grid=(g,),
        in_specs=[
            pl.BlockSpec((bn, D), lambda i: (i, 0)),
            pl.BlockSpec((br, 800, D), lambda i: (i, 0, 0)),
            pl.BlockSpec((D, D), lambda i: (0, 0)),
            pl.BlockSpec((1, D), lambda i: (0, 0)),
            pl.BlockSpec((D, D), lambda i: (0, 0)),
            pl.BlockSpec((1, D), lambda i: (0, 0)),
        ],
        out_specs=[
            pl.BlockSpec((bn, DH), lambda i: (i, 0)),
            pl.BlockSpec((bn, DH), lambda i: (i, 0)),
            pl.BlockSpec((br, 800), lambda i: (i, 0)),
        ],
        out_shape=[
            jax.ShapeDtypeStruct((N, DH), jnp.float32),
            jax.ShapeDtypeStruct((N, DH), jnp.float32),
            jax.ShapeDtypeStruct((E // 800, 800), jnp.float32),
        ],
    )(x, ea3, w, b2, we, be2)
    return ha, hb, ew


def _rsqrt16(d):
    xi = plsc.bitcast(d, jnp.int32)
    yi = jnp.int32(0x5F3759DF) - lax.shift_right_logical(xi, 1)
    y = plsc.bitcast(yi, jnp.float32)
    for _ in range(3):
        y = y * (1.5 - 0.5 * d * y * y)
    return y


_SC_MESH = plsc.VectorSubcoreMesh(
    core_axis_name="c", subcore_axis_name="s", num_cores=2, num_subcores=NS)


@functools.partial(
    pl.kernel,
    out_type=jax.ShapeDtypeStruct((N, D), jnp.float32),
    mesh=_SC_MESH,
    scratch_types=[
        pltpu.VMEM((80, 128), jnp.float32),
        pltpu.VMEM((CH, DH), jnp.float32),
        pltpu.VMEM((CH, DH), jnp.float32),
        pltpu.VMEM((B_CH, CH), jnp.int32),
        pltpu.VMEM((B_CH, CH), jnp.int32),
        pltpu.VMEM((B_CH, CH), jnp.float32),
        pltpu.VMEM((80,), jnp.int32),
        pltpu.VMEM((1, DH), jnp.float32),
        pltpu.VMEM_SHARED((80, 128), jnp.float32),
        pltpu.VMEM_SHARED((80, 128), jnp.float32),
        pltpu.VMEM_SHARED((N_PAD, DH), jnp.float32),
        pltpu.SemaphoreType.DMA,
        pltpu.SemaphoreType.DMA,
        pltpu.SemaphoreType.DMA,
        pltpu.SemaphoreType.DMA,
    ],
    compiler_params=pltpu.CompilerParams(needs_layout_passes=False),
)
def _sc_kernel(ha_hbm, hb_hbm, row_hbm, col_hbm, ew_hbm, bias_hbm, out_hbm,
               dd_v, buf0, buf1, rowb, colb, ewb, iota80, bias_v,
               deg_sh, dis_sh, acc_sh, sem_g0, sem_g1, sem_s0, sem_s1):
    cid = lax.axis_index("c")
    sid = lax.axis_index("s")
    zero16 = jnp.zeros((16,), jnp.float32)

    pltpu.sync_copy(bias_hbm.at[cid], bias_v)

    def _zrow(r, _):
        for j in range(8):
            dd_v[r, pl.ds(j * 16, 16)] = zero16
        return 0
    lax.fori_loop(0, 80, _zrow, 0)

    def _blk_a(b, _):
        blk = sid * NBLK + b
        pltpu.sync_copy(col_hbm.at[blk], colb)
        pltpu.sync_copy(ew_hbm.at[blk], ewb)

        def _ch(kk, _):
            for p in range(5):
                c16 = colb[kk, pl.ds(p * 16, 16)]
                w16 = ewb[kk, pl.ds(p * 16, 16)]
                plsc.addupdate_scatter(
                    dd_v, [lax.shift_right_logical(c16, 7),
                           lax.bitwise_and(c16, 127)], w16)
            return 0
        lax.fori_loop(0, B_CH, _ch, 0)
        return 0
    lax.fori_loop(0, NBLK, _blk_a, 0)

    def _iota(i, _):
        iota80[pl.ds(i * 16, 16)] = lax.iota(jnp.int32, 16) + i * 16
        return 0
    lax.fori_loop(0, 5, _iota, 0)

    @pl.when(sid == 0)
    def _():
        pltpu.sync_copy(dd_v, deg_sh)
    plsc.subcore_barrier()

    @pl.when(sid != 0)
    def _():
        pltpu.sync_copy(dd_v, deg_sh.at[iota80], add=True)
    plsc.subcore_barrier()

    @pl.when(sid < 10)
    def _():
        r0 = sid * 8
        pltpu.sync_copy(deg_sh.at[pl.ds(r0, 8)], buf1.at[pl.ds(0, 8)])
        for i in range(8):
            for j in range(8):
                d = buf1[i, pl.ds(j * 16, 16)]
                y = _rsqrt16(d)
                buf1[i, pl.ds(j * 16, 16)] = jnp.where(d > 0.0, y, 0.0)
        pltpu.sync_copy(buf1.at[pl.ds(0, 8)], dis_sh.at[pl.ds(r0, 8)])
    plsc.subcore_barrier()
    pltpu.sync_copy(dis_sh, dd_v)

    def _zbuf(r, _):
        for j in range(8):
            buf0[r, pl.ds(j * 16, 16)] = zero16
        return 0
    lax.fori_loop(0, CH, _zbuf, 0)
    for k in range(8):
        pltpu.sync_copy(buf0, acc_sh.at[pl.ds(sid * 640 + k * 80, 80)])
    plsc.subcore_barrier()

    def _issue(idx_row, dstbuf, sem):
        @pl.when(cid == 0)
        def _():
            pltpu.async_copy(ha_hbm.at[idx_row], dstbuf, sem)

        @pl.when(cid != 0)
        def _():
            pltpu.async_copy(hb_hbm.at[idx_row], dstbuf, sem)

    def _scales(kk):
        out = []
        for p in range(5):
            r16 = rowb[kk, pl.ds(p * 16, 16)]
            c16 = colb[kk, pl.ds(p * 16, 16)]
            dr = plsc.load_gather(
                dd_v, [lax.shift_right_logical(r16, 7),
                       lax.bitwise_and(r16, 127)])
            dc = plsc.load_gather(
                dd_v, [lax.shift_right_logical(c16, 7),
                       lax.bitwise_and(c16, 127)])
            out.append(ewb[kk, pl.ds(p * 16, 16)] * dr * dc)
        return out

    def _scale_rows(bufx, scales):
        for p in range(5):
            s16 = scales[p]
            for i in range(16):
                s = s16[i]
                for j in range(8):
                    bufx[p * 16 + i, pl.ds(j * 16, 16)] = (
                        bufx[p * 16 + i, pl.ds(j * 16, 16)] * s)

    def _blk_b(b, _):
        @pl.when(b > 0)
        def _():
            pltpu.make_async_copy(buf0, acc_sh.at[colb.at[0]], sem_s0).wait()
            pltpu.make_async_copy(buf1, acc_sh.at[colb.at[0]], sem_s1).wait()
        blk = sid * NBLK + b
        pltpu.sync_copy(row_hbm.at[blk], rowb)
        pltpu.sync_copy(col_hbm.at[blk], colb)
        pltpu.sync_copy(ew_hbm.at[blk], ewb)
        _issue(rowb.at[0], buf0, sem_g0)

        def _pair(p, _):
            k0 = 2 * p
            k1 = 2 * p + 1

            @pl.when(p > 0)
            def _():
                pltpu.make_async_copy(
                    buf1, acc_sh.at[colb.at[k1]], sem_s1).wait()
            _issue(rowb.at[k1], buf1, sem_g1)
            sc0 = _scales(k0)
            pltpu.make_async_copy(ha_hbm.at[rowb.at[k0]], buf0, sem_g0).wait()
            _scale_rows(buf0, sc0)
            pltpu.async_copy(buf0, acc_sh.at[colb.at[k0]], sem_s0, add=True)
            sc1 = _scales(k1)
            pltpu.make_async_copy(ha_hbm.at[rowb.at[k1]], buf1, sem_g1).wait()
            _scale_rows(buf1, sc1)
            pltpu.make_async_copy(
                buf0, acc_sh.at[colb.at[k0]], sem_s0).wait()
            _issue(rowb.at[k0 + 2], buf0, sem_g0)
            pltpu.async_copy(buf1, acc_sh.at[colb.at[k1]], sem_s1, add=True)
            return 0
        lax.fori_loop(0, (B_CH - 1) // 2, _pair, 0)

        sc = _scales(B_CH - 1)
        pltpu.make_async_copy(
            ha_hbm.at[rowb.at[B_CH - 1]], buf0, sem_g0).wait()
        _scale_rows(buf0, sc)
        pltpu.async_copy(
            buf0, acc_sh.at[colb.at[B_CH - 1]], sem_s0, add=True)
        return 0
    lax.fori_loop(0, NBLK, _blk_b, 0)
    pltpu.make_async_copy(buf0, acc_sh.at[colb.at[0]], sem_s0).wait()
    pltpu.make_async_copy(buf1, acc_sh.at[colb.at[0]], sem_s1).wait()
    plsc.subcore_barrier()

    def _wb(r, _):
        ck = r * NS + sid

        @pl.when(ck < NWCH)
        def _():
            pltpu.sync_copy(acc_sh.at[pl.ds(ck * CH, CH)], buf0)

            def _wrow(i, _):
                for j in range(8):
                    v = (buf0[i, pl.ds(j * 16, 16)]
                         + bias_v[0, pl.ds(j * 16, 16)])
                    buf0[i, pl.ds(j * 16, 16)] = jnp.maximum(v, 0.0)
                return 0
            lax.fori_loop(0, CH, _wrow, 0)
            pltpu.sync_copy(
                buf0,
                out_hbm.at[pl.ds(ck * CH, CH), pl.ds(cid * DH, DH)])
        return 0
    lax.fori_loop(0, (NWCH + NS - 1) // NS, _wb, 0)


def kernel(x, edge_index, edge_attr, W, b_lin, We, be, bias):
    row = edge_index[0]
    col = edge_index[1]
    ha, hb, ew = _tc_call(x, edge_attr.reshape(E // 800, 800, D), W,
                          b_lin.reshape(1, D), We, be.reshape(1, D))
    return _sc_kernel(ha, hb, row.reshape(NS * NBLK, B_CH, CH),
                      col.reshape(NS * NBLK, B_CH, CH),
                      ew.reshape(NS * NBLK, B_CH, CH),
                      bias.reshape(2, 1, DH))


# speedup vs baseline: 10.7291x; 1.0499x over previous
import functools

import jax
import jax.numpy as jnp
from jax import lax
from jax.experimental import pallas as pl
from jax.experimental.pallas import tpu as pltpu
from jax.experimental.pallas import tpu_sc as plsc

N = 10000
E = 160000
D = 256
DH = 128
NS = 16
N_PAD = 10240
EPT = E // NS
CH = 80
NCHUNK = EPT // CH
B_CH = 25
NBLK = NCHUNK // B_CH
NWCH = N // CH


def _tc_body(x_ref, ea_ref, w_ref, b_ref, we_ref, be_ref,
             ha_ref, hb_ref, ew_ref):
    h = lax.dot_general(
        x_ref[...], w_ref[...], (((1,), (1,)), ((), ())),
        preferred_element_type=jnp.float32) + b_ref[0]
    ha_ref[...] = h[:, :DH]
    hb_ref[...] = h[:, DH:]
    wbar = jnp.mean(we_ref[...], axis=0)
    bebar = jnp.mean(be_ref[...])
    logits = jnp.sum(ea_ref[...] * wbar, axis=-1) + bebar
    ew_ref[...] = jnp.clip(jax.nn.sigmoid(logits), 0.0001, 1.0)


def _tc_call(x, ea3, w, b2, we, be2):
    g = 25
    bn = N // g
    br = (E // 800) // g
    ha, hb, ew = pl.pallas_call(
        _tc_body,
        grid=(g,),
        in_specs=[
            pl.BlockSpec((bn, D), lambda i: (i, 0)),
            pl.BlockSpec((br, 800, D), lambda i: (i, 0, 0)),
            pl.BlockSpec((D, D), lambda i: (0, 0)),
            pl.BlockSpec((1, D), lambda i: (0, 0)),
            pl.BlockSpec((D, D), lambda i: (0, 0)),
            pl.BlockSpec((1, D), lambda i: (0, 0)),
        ],
        out_specs=[
            pl.BlockSpec((bn, DH), lambda i: (i, 0)),
            pl.BlockSpec((bn, DH), lambda i: (i, 0)),
            pl.BlockSpec((br, 800), lambda i: (i, 0)),
        ],
        out_shape=[
            jax.ShapeDtypeStruct((N, DH), jnp.float32),
            jax.ShapeDtypeStruct((N, DH), jnp.float32),
            jax.ShapeDtypeStruct((E // 800, 800), jnp.float32),
        ],
    )(x, ea3, w, b2, we, be2)
    return ha, hb, ew


def _rsqrt16(d):
    xi = plsc.bitcast(d, jnp.int32)
    yi = jnp.int32(0x5F3759DF) - lax.shift_right_logical(xi, 1)
    y = plsc.bitcast(yi, jnp.float32)
    for _ in range(3):
        y = y * (1.5 - 0.5 * d * y * y)
    return y


_SC_MESH = plsc.VectorSubcoreMesh(
    core_axis_name="c", subcore_axis_name="s", num_cores=2, num_subcores=NS)


@functools.partial(
    pl.kernel,
    out_type=jax.ShapeDtypeStruct((N, D), jnp.float32),
    mesh=_SC_MESH,
    scratch_types=[
        pltpu.VMEM((80, 128), jnp.float32),
        pltpu.VMEM((CH, DH), jnp.float32),
        pltpu.VMEM((CH, DH), jnp.float32),
        pltpu.VMEM((B_CH, CH), jnp.int32),
        pltpu.VMEM((B_CH, CH), jnp.int32),
        pltpu.VMEM((B_CH, CH), jnp.float32),
        pltpu.VMEM((80,), jnp.int32),
        pltpu.VMEM((1, DH), jnp.float32),
        pltpu.VMEM_SHARED((80, 128), jnp.float32),
        pltpu.VMEM_SHARED((80, 128), jnp.float32),
        pltpu.VMEM_SHARED((N_PAD, DH), jnp.float32),
        pltpu.SemaphoreType.DMA,
        pltpu.SemaphoreType.DMA,
        pltpu.SemaphoreType.DMA,
        pltpu.SemaphoreType.DMA,
    ],
    compiler_params=pltpu.CompilerParams(needs_layout_passes=False),
)
def _sc_kernel(ha_hbm, hb_hbm, row_hbm, col_hbm, ew_hbm, bias_hbm, out_hbm,
               dd_v, buf0, buf1, rowb, colb, ewb, iota80, bias_v,
               deg_sh, dis_sh, acc_sh, sem_g0, sem_g1, sem_s0, sem_s1):
    cid = lax.axis_index("c")
    sid = lax.axis_index("s")
    zero16 = jnp.zeros((16,), jnp.float32)

    pltpu.sync_copy(bias_hbm.at[cid], bias_v)

    def _zrow(r, _):
        for j in range(8):
            dd_v[r, pl.ds(j * 16, 16)] = zero16
        return 0
    lax.fori_loop(0, 80, _zrow, 0)

    def _blk_a(b, _):
        blk = sid * NBLK + b
        pltpu.sync_copy(col_hbm.at[blk], colb)
        pltpu.sync_copy(ew_hbm.at[blk], ewb)

        def _ch(kk, _):
            for p in range(5):
                c16 = colb[kk, pl.ds(p * 16, 16)]
                w16 = ewb[kk, pl.ds(p * 16, 16)]
                plsc.addupdate_scatter(
                    dd_v, [lax.shift_right_logical(c16, 7),
                           lax.bitwise_and(c16, 127)], w16)
            return 0
        lax.fori_loop(0, B_CH, _ch, 0)
        return 0
    lax.fori_loop(0, NBLK, _blk_a, 0)

    def _iota(i, _):
        iota80[pl.ds(i * 16, 16)] = lax.iota(jnp.int32, 16) + i * 16
        return 0
    lax.fori_loop(0, 5, _iota, 0)

    @pl.when(sid == 0)
    def _():
        pltpu.sync_copy(dd_v, deg_sh)
    plsc.subcore_barrier()

    @pl.when(sid != 0)
    def _():
        pltpu.sync_copy(dd_v, deg_sh.at[iota80], add=True)
    plsc.subcore_barrier()

    @pl.when(sid < 10)
    def _():
        r0 = sid * 8
        pltpu.sync_copy(deg_sh.at[pl.ds(r0, 8)], buf1.at[pl.ds(0, 8)])
        for i in range(8):
            for j in range(8):
                d = buf1[i, pl.ds(j * 16, 16)]
                y = _rsqrt16(d)
                buf1[i, pl.ds(j * 16, 16)] = jnp.where(d > 0.0, y, 0.0)
        pltpu.sync_copy(buf1.at[pl.ds(0, 8)], dis_sh.at[pl.ds(r0, 8)])
    plsc.subcore_barrier()
    pltpu.sync_copy(dis_sh, dd_v)

    def _zbuf(r, _):
        for j in range(8):
            buf0[r, pl.ds(j * 16, 16)] = zero16
        return 0
    lax.fori_loop(0, CH, _zbuf, 0)
    for k in range(8):
        pltpu.sync_copy(buf0, acc_sh.at[pl.ds(sid * 640 + k * 80, 80)])
    plsc.subcore_barrier()

    def _issue(idx_row, dstbuf, sem):
        @pl.when(cid == 0)
        def _():
            pltpu.async_copy(ha_hbm.at[idx_row], dstbuf, sem)

        @pl.when(cid != 0)
        def _():
            pltpu.async_copy(hb_hbm.at[idx_row], dstbuf, sem)

    def _scales(kk):
        out = []
        for p in range(5):
            r16 = rowb[kk, pl.ds(p * 16, 16)]
            c16 = colb[kk, pl.ds(p * 16, 16)]
            dr = plsc.load_gather(
                dd_v, [lax.shift_right_logical(r16, 7),
                       lax.bitwise_and(r16, 127)])
            dc = plsc.load_gather(
                dd_v, [lax.shift_right_logical(c16, 7),
                       lax.bitwise_and(c16, 127)])
            out.append(ewb[kk, pl.ds(p * 16, 16)] * dr * dc)
        return out

    def _scale_rows(bufx, scales):
        for p in range(5):
            s16 = scales[p]
            for i in range(16):
                s = s16[i]
                for j in range(8):
                    bufx[p * 16 + i, pl.ds(j * 16, 16)] = (
                        bufx[p * 16 + i, pl.ds(j * 16, 16)] * s)

    def _blk_b(b, _):
        @pl.when(b > 0)
        def _():
            pltpu.make_async_copy(buf0, acc_sh.at[colb.at[0]], sem_s0).wait()
            pltpu.make_async_copy(buf1, acc_sh.at[colb.at[0]], sem_s1).wait()
        blk = sid * NBLK + b
        pltpu.sync_copy(row_hbm.at[blk], rowb)
        pltpu.sync_copy(col_hbm.at[blk], colb)
        pltpu.sync_copy(ew_hbm.at[blk], ewb)
        _issue(rowb.at[0], buf0, sem_g0)

        def _pair(p, _):
            k0 = 2 * p
            k1 = 2 * p + 1

            @pl.when(p > 0)
            def _():
                pltpu.make_async_copy(
                    buf1, acc_sh.at[colb.at[k1]], sem_s1).wait()
            _issue(rowb.at[k1], buf1, sem_g1)
            sc0 = _scales(k0)
            pltpu.make_async_copy(ha_hbm.at[rowb.at[k0]], buf0, sem_g0).wait()
            _scale_rows(buf0, sc0)
            pltpu.async_copy(buf0, acc_sh.at[colb.at[k0]], sem_s0, add=True)
            sc1 = _scales(k1)
            pltpu.make_async_copy(
                buf0, acc_sh.at[colb.at[k0]], sem_s0).wait()
            _issue(rowb.at[k0 + 2], buf0, sem_g0)
            pltpu.make_async_copy(ha_hbm.at[rowb.at[k1]], buf1, sem_g1).wait()
            _scale_rows(buf1, sc1)
            pltpu.async_copy(buf1, acc_sh.at[colb.at[k1]], sem_s1, add=True)
            return 0
        lax.fori_loop(0, (B_CH - 1) // 2, _pair, 0)

        sc = _scales(B_CH - 1)
        pltpu.make_async_copy(
            ha_hbm.at[rowb.at[B_CH - 1]], buf0, sem_g0).wait()
        _scale_rows(buf0, sc)
        pltpu.async_copy(
            buf0, acc_sh.at[colb.at[B_CH - 1]], sem_s0, add=True)
        return 0
    lax.fori_loop(0, NBLK, _blk_b, 0)
    pltpu.make_async_copy(buf0, acc_sh.at[colb.at[0]], sem_s0).wait()
    pltpu.make_async_copy(buf1, acc_sh.at[colb.at[0]], sem_s1).wait()
    plsc.subcore_barrier()

    def _wb(r, _):
        ck = r * NS + sid

        @pl.when(ck < NWCH)
        def _():
            pltpu.sync_copy(acc_sh.at[pl.ds(ck * CH, CH)], buf0)

            def _wrow(i, _):
                for j in range(8):
                    v = (buf0[i, pl.ds(j * 16, 16)]
                         + bias_v[0, pl.ds(j * 16, 16)])
                    buf0[i, pl.ds(j * 16, 16)] = jnp.maximum(v, 0.0)
                return 0
            lax.fori_loop(0, CH, _wrow, 0)
            pltpu.sync_copy(
                buf0,
                out_hbm.at[pl.ds(ck * CH, CH), pl.ds(cid * DH, DH)])
        return 0
    lax.fori_loop(0, (NWCH + NS - 1) // NS, _wb, 0)


def kernel(x, edge_index, edge_attr, W, b_lin, We, be, bias):
    row = edge_index[0]
    col = edge_index[1]
    ha, hb, ew = _tc_call(x, edge_attr.reshape(E // 800, 800, D), W,
                          b_lin.reshape(1, D), We, be.reshape(1, D))
    return _sc_kernel(ha, hb, row.reshape(NS * NBLK, B_CH, CH),
                      col.reshape(NS * NBLK, B_CH, CH),
                      ew.reshape(NS * NBLK, B_CH, CH),
                      bias.reshape(2, 1, DH))
